# baseline (speedup 1.0000x reference)
import functools

import jax
import jax.numpy as jnp
from jax import lax
from jax.experimental import pallas as pl
from jax.experimental.pallas import tpu as pltpu
from jax.experimental.pallas import tpu_sc as plsc


def _tc_body(Ms, sem_ref, sp_ref, tp_ref, wb_ref, gidx_ref, sval_ref):
    b = pl.program_id(0)
    j = pl.program_id(1)

    @pl.when(j == 0)
    def _():
        logits = sem_ref[0]
        mx = jnp.max(logits, axis=0, keepdims=True)
        e = jnp.exp(logits - mx)
        p = e / jnp.sum(e, axis=0, keepdims=True)
        s = jnp.sum(p * wb_ref[...], axis=0)
        sval_ref[...] = 1.0 / (1.0 + jnp.exp(-s))

    t = tp_ref[0]
    s = sp_ref[0]
    s2h = 0.5 * jnp.sum(s * s, axis=0)
    st = lax.dot_general(s, t, (((0,), (0,)), ((), ())),
                         preferred_element_type=jnp.float32)
    e = s2h[:, None] - st
    arg = jnp.argmin(e, axis=0).astype(jnp.int32)
    gidx_ref[...] = arg + b * Ms


def _tc_stage(sem_logits, source_pos, target_pos, wb, tn):
    B, K, Ms = sem_logits.shape
    Nt = target_pos.shape[2]
    nb = Nt // tn
    grid = (B, nb)
    gidx, sval = pl.pallas_call(
        functools.partial(_tc_body, Ms),
        grid=grid,
        in_specs=[
            pl.BlockSpec((1, K, Ms), lambda b, j: (b, 0, 0)),
            pl.BlockSpec((1, 3, Ms), lambda b, j: (b, 0, 0)),
            pl.BlockSpec((1, 3, tn), lambda b, j: (b, 0, j)),
            pl.BlockSpec((K, 1), lambda b, j: (0, 0)),
        ],
        out_specs=[
            pl.BlockSpec((tn,), lambda b, j: (b * nb + j,)),
            pl.BlockSpec((Ms,), lambda b, j: (b,)),
        ],
        out_shape=[
            jax.ShapeDtypeStruct((B * Nt,), jnp.int32),
            jax.ShapeDtypeStruct((B * Ms,), jnp.float32),
        ],
    )(sem_logits, source_pos, target_pos, wb)
    return gidx, sval


def _sc_gather(sval_flat, gidx_flat):
    info = plsc.get_sparse_core_info()
    NC, NS = info.num_cores, info.num_subcores
    NW = NC * NS
    N = gidx_flat.shape[0]
    per = N // NW
    CW = 128
    C = per // CW
    gidx3 = gidx_flat.reshape(NW, C, CW)
    mesh = plsc.VectorSubcoreMesh(core_axis_name="c", subcore_axis_name="s")

    @functools.partial(
        pl.kernel,
        mesh=mesh,
        out_type=jax.ShapeDtypeStruct((NW, C, CW), jnp.float32),
        scratch_types=[
            pltpu.VMEM((C, CW), jnp.int32),
            pltpu.VMEM((C, CW), jnp.float32),
            pltpu.SemaphoreType.DMA,
        ],
    )
    def k(sval_hbm, idx_hbm, out_hbm, idx_v, out_v, sem):
        wid = lax.axis_index("s") * NC + lax.axis_index("c")
        pltpu.sync_copy(idx_hbm.at[wid], idx_v)
        copies = [
            pltpu.async_copy(sval_hbm.at[idx_v.at[c]], out_v.at[c], sem)
            for c in range(C)
        ]
        for cp in copies:
            cp.wait()
        pltpu.sync_copy(out_v, out_hbm.at[wid])

    return k(sval_flat, gidx3).reshape(N)


def kernel(sem_logits, source_pos, target_pos, conv_w, conv_b):
    B, K, Ms = sem_logits.shape
    Nt = target_pos.shape[2]
    wb = (conv_w[0, :, 0] + conv_b[0]).reshape(K, 1)
    h = B // 2
    g0, v0 = _tc_stage(sem_logits[:h], source_pos[:h], target_pos[:h], wb,
                       tn=2048)
    g1, v1 = _tc_stage(sem_logits[h:], source_pos[h:], target_pos[h:], wb,
                       tn=2048)
    out0 = _sc_gather(v0, g0)
    out1 = _sc_gather(v1, g1)
    return jnp.concatenate([out0, out1]).reshape(B, 1, Nt)


# speedup vs baseline: 1.0875x; 1.0875x over previous
import functools

import jax
import jax.numpy as jnp
from jax import lax
from jax.experimental import pallas as pl
from jax.experimental.pallas import tpu as pltpu
from jax.experimental.pallas import tpu_sc as plsc


def _tc_body(Ms, sem_ref, sp_ref, tp_ref, wb_ref, gidx_ref, sval_ref):
    b = pl.program_id(0)
    j = pl.program_id(1)

    @pl.when(j == 0)
    def _():
        logits = sem_ref[0]
        mx = jnp.max(logits, axis=0, keepdims=True)
        e = jnp.exp(logits - mx)
        p = e / jnp.sum(e, axis=0, keepdims=True)
        s = jnp.sum(p * wb_ref[...], axis=0)
        sval_ref[...] = 1.0 / (1.0 + jnp.exp(-s))

    t = tp_ref[0]
    s = sp_ref[0]
    s2h = 0.5 * jnp.sum(s * s, axis=0)
    st = lax.dot_general(s, t, (((0,), (0,)), ((), ())),
                         preferred_element_type=jnp.float32)
    e = s2h[:, None] - st
    arg = jnp.argmin(e, axis=0).astype(jnp.int32)
    gidx_ref[...] = arg + b * Ms


def _tc_stage(sem_logits, source_pos, target_pos, wb, tn):
    B, K, Ms = sem_logits.shape
    Nt = target_pos.shape[2]
    nb = Nt // tn
    grid = (B, nb)
    gidx, sval = pl.pallas_call(
        functools.partial(_tc_body, Ms),
        grid=grid,
        in_specs=[
            pl.BlockSpec((1, K, Ms), lambda b, j: (b, 0, 0)),
            pl.BlockSpec((1, 3, Ms), lambda b, j: (b, 0, 0)),
            pl.BlockSpec((1, 3, tn), lambda b, j: (b, 0, j)),
            pl.BlockSpec((K, 1), lambda b, j: (0, 0)),
        ],
        out_specs=[
            pl.BlockSpec((tn,), lambda b, j: (b * nb + j,)),
            pl.BlockSpec((Ms,), lambda b, j: (b,)),
        ],
        out_shape=[
            jax.ShapeDtypeStruct((B * Nt,), jnp.int32),
            jax.ShapeDtypeStruct((B * Ms,), jnp.float32),
        ],
    )(sem_logits, source_pos, target_pos, wb)
    return gidx, sval


def _sc_gather(sval_flat, gidx_flat):
    info = plsc.get_sparse_core_info()
    NC, NS = info.num_cores, info.num_subcores
    NW = NC * NS
    N = gidx_flat.shape[0]
    per = N // NW
    CW = 128
    C = per // CW
    gidx3 = gidx_flat.reshape(NW, C, CW)
    mesh = plsc.VectorSubcoreMesh(core_axis_name="c", subcore_axis_name="s")

    @functools.partial(
        pl.kernel,
        mesh=mesh,
        out_type=jax.ShapeDtypeStruct((NW, C, CW), jnp.float32),
        scratch_types=[
            pltpu.VMEM((C, CW), jnp.int32),
            pltpu.VMEM((C, CW), jnp.float32),
            pltpu.SemaphoreType.DMA,
        ],
    )
    def k(sval_hbm, idx_hbm, out_hbm, idx_v, out_v, sem):
        wid = lax.axis_index("s") * NC + lax.axis_index("c")
        pltpu.sync_copy(idx_hbm.at[wid], idx_v)
        copies = [
            pltpu.async_copy(sval_hbm.at[idx_v.at[c]], out_v.at[c], sem)
            for c in range(C)
        ]
        for cp in copies:
            cp.wait()
        pltpu.sync_copy(out_v, out_hbm.at[wid])

    return k(sval_flat, gidx3).reshape(N)


def kernel(sem_logits, source_pos, target_pos, conv_w, conv_b):
    B, K, Ms = sem_logits.shape
    Nt = target_pos.shape[2]
    wb = (conv_w[0, :, 0] + conv_b[0]).reshape(K, 1)
    gidx_flat, sval_flat = _tc_stage(sem_logits, source_pos, target_pos, wb,
                                     tn=4096)
    gate_flat = _sc_gather(sval_flat, gidx_flat)
    return gate_flat.reshape(B, 1, Nt)


# speedup vs baseline: 1.1568x; 1.0637x over previous
import functools

import jax
import jax.numpy as jnp
from jax import lax
from jax.experimental import pallas as pl
from jax.experimental.pallas import tpu as pltpu
from jax.experimental.pallas import tpu_sc as plsc


def _tc_body(Ms, sem_ref, sp_ref, tp_ref, wb_ref, gidx_ref, sval_ref):
    b = pl.program_id(0)
    j = pl.program_id(1)

    @pl.when(j == 0)
    def _():
        logits = sem_ref[0]
        mx = jnp.max(logits, axis=0, keepdims=True)
        e = jnp.exp(logits - mx)
        p = e / jnp.sum(e, axis=0, keepdims=True)
        s = jnp.sum(p * wb_ref[...], axis=0)
        sval_ref[...] = 1.0 / (1.0 + jnp.exp(-s))

    t = tp_ref[0]
    s = sp_ref[0]
    s2h = 0.5 * jnp.sum(s * s, axis=0)
    st = lax.dot_general(s, t, (((0,), (0,)), ((), ())),
                         preferred_element_type=jnp.float32)
    e = s2h[:, None] - st
    arg = jnp.argmin(e, axis=0).astype(jnp.int32)
    gidx_ref[...] = arg + b * Ms


def _tc_stage(sem_logits, source_pos, target_pos, wb, tn):
    B, K, Ms = sem_logits.shape
    Nt = target_pos.shape[2]
    nb = Nt // tn
    grid = (B, nb)
    gidx, sval = pl.pallas_call(
        functools.partial(_tc_body, Ms),
        grid=grid,
        in_specs=[
            pl.BlockSpec((1, K, Ms), lambda b, j: (b, 0, 0)),
            pl.BlockSpec((1, 3, Ms), lambda b, j: (b, 0, 0)),
            pl.BlockSpec((1, 3, tn), lambda b, j: (b, 0, j)),
            pl.BlockSpec((K, 1), lambda b, j: (0, 0)),
        ],
        out_specs=[
            pl.BlockSpec((tn,), lambda b, j: (b * nb + j,)),
            pl.BlockSpec((Ms,), lambda b, j: (b,)),
        ],
        out_shape=[
            jax.ShapeDtypeStruct((B * Nt,), jnp.int32),
            jax.ShapeDtypeStruct((B * Ms,), jnp.float32),
        ],
    )(sem_logits, source_pos, target_pos, wb)
    return gidx, sval


def _sc_gather(sval_flat, gidx_flat):
    info = plsc.get_sparse_core_info()
    NC, NS = info.num_cores, info.num_subcores
    NW = NC * NS
    N = gidx_flat.shape[0]
    per = N // NW
    CW = 128
    C = per // CW
    gidx3 = gidx_flat.reshape(NW, C, CW)
    mesh = plsc.VectorSubcoreMesh(core_axis_name="c", subcore_axis_name="s")

    S = sval_flat.shape[0]

    @functools.partial(
        pl.kernel,
        mesh=mesh,
        out_type=jax.ShapeDtypeStruct((NW, C, CW), jnp.float32),
        scratch_types=[
            pltpu.VMEM((C, CW), jnp.int32),
            pltpu.VMEM((C, CW), jnp.float32),
            pltpu.VMEM_SHARED((S,), jnp.float32),
            pltpu.SemaphoreType.DMA,
        ],
    )
    def k(sval_hbm, idx_hbm, out_hbm, idx_v, out_v, table_sh, sem):
        sid = lax.axis_index("s")
        wid = sid * NC + lax.axis_index("c")

        @pl.when(sid == 0)
        def _():
            pltpu.sync_copy(sval_hbm, table_sh)

        pltpu.sync_copy(idx_hbm.at[wid], idx_v)
        plsc.subcore_barrier()
        copies = [
            pltpu.async_copy(table_sh.at[idx_v.at[c]], out_v.at[c], sem)
            for c in range(C)
        ]
        for cp in copies:
            cp.wait()
        pltpu.sync_copy(out_v, out_hbm.at[wid])

    return k(sval_flat, gidx3).reshape(N)


def kernel(sem_logits, source_pos, target_pos, conv_w, conv_b):
    B, K, Ms = sem_logits.shape
    Nt = target_pos.shape[2]
    wb = (conv_w[0, :, 0] + conv_b[0]).reshape(K, 1)
    gidx_flat, sval_flat = _tc_stage(sem_logits, source_pos, target_pos, wb,
                                     tn=4096)
    gate_flat = _sc_gather(sval_flat, gidx_flat)
    return gate_flat.reshape(B, 1, Nt)


# speedup vs baseline: 1.2547x; 1.0846x over previous
import functools

import jax
import jax.numpy as jnp
from jax import lax
from jax.experimental import pallas as pl
from jax.experimental.pallas import tpu as pltpu
from jax.experimental.pallas import tpu_sc as plsc


def _tc_body(Ms, sem_ref, sp_ref, tp_ref, wb_ref, gidx_ref, sval_ref):
    b = pl.program_id(0)
    j = pl.program_id(1)

    @pl.when(j == 0)
    def _():
        logits = sem_ref[0]
        mx = jnp.max(logits, axis=0, keepdims=True)
        e = jnp.exp(logits - mx)
        p = e / jnp.sum(e, axis=0, keepdims=True)
        s = jnp.sum(p * wb_ref[...], axis=0)
        sval_ref[...] = 1.0 / (1.0 + jnp.exp(-s))

    t = tp_ref[0]
    s = sp_ref[0]
    s2h = 0.5 * jnp.sum(s * s, axis=0)
    lhs = jnp.concatenate([-s, s2h[None, :]], axis=0)
    ones = jnp.ones((1, t.shape[1]), jnp.float32)
    rhs = jnp.concatenate([t, ones], axis=0)
    e = lax.dot_general(lhs, rhs, (((0,), (0,)), ((), ())),
                        preferred_element_type=jnp.float32)
    arg = jnp.argmin(e, axis=0).astype(jnp.int32)
    gidx_ref[...] = arg + b * Ms


def _tc_stage(sem_logits, source_pos, target_pos, wb, tn):
    B, K, Ms = sem_logits.shape
    Nt = target_pos.shape[2]
    nb = Nt // tn
    grid = (B, nb)
    gidx, sval = pl.pallas_call(
        functools.partial(_tc_body, Ms),
        grid=grid,
        in_specs=[
            pl.BlockSpec((1, K, Ms), lambda b, j: (b, 0, 0)),
            pl.BlockSpec((1, 3, Ms), lambda b, j: (b, 0, 0)),
            pl.BlockSpec((1, 3, tn), lambda b, j: (b, 0, j)),
            pl.BlockSpec((K, 1), lambda b, j: (0, 0)),
        ],
        out_specs=[
            pl.BlockSpec((tn,), lambda b, j: (b * nb + j,)),
            pl.BlockSpec((Ms,), lambda b, j: (b,)),
        ],
        out_shape=[
            jax.ShapeDtypeStruct((B * Nt,), jnp.int32),
            jax.ShapeDtypeStruct((B * Ms,), jnp.float32),
        ],
    )(sem_logits, source_pos, target_pos, wb)
    return gidx, sval


def _sc_gather(sval_flat, gidx_flat):
    info = plsc.get_sparse_core_info()
    NC, NS = info.num_cores, info.num_subcores
    NW = NC * NS
    N = gidx_flat.shape[0]
    per = N // NW
    CW = 128
    C = per // CW
    gidx3 = gidx_flat.reshape(NW, C, CW)
    mesh = plsc.VectorSubcoreMesh(core_axis_name="c", subcore_axis_name="s")

    S = sval_flat.shape[0]

    @functools.partial(
        pl.kernel,
        mesh=mesh,
        out_type=jax.ShapeDtypeStruct((NW, C, CW), jnp.float32),
        scratch_types=[
            pltpu.VMEM((C, CW), jnp.int32),
            pltpu.VMEM((C, CW), jnp.float32),
            pltpu.VMEM_SHARED((S,), jnp.float32),
            pltpu.SemaphoreType.DMA,
        ],
    )
    def k(sval_hbm, idx_hbm, out_hbm, idx_v, out_v, table_sh, sem):
        sid = lax.axis_index("s")
        wid = sid * NC + lax.axis_index("c")

        @pl.when(sid == 0)
        def _():
            pltpu.sync_copy(sval_hbm, table_sh)

        pltpu.sync_copy(idx_hbm.at[wid], idx_v)
        plsc.subcore_barrier()
        copies = [
            pltpu.async_copy(table_sh.at[idx_v.at[c]], out_v.at[c], sem)
            for c in range(C)
        ]
        for cp in copies:
            cp.wait()
        pltpu.sync_copy(out_v, out_hbm.at[wid])

    return k(sval_flat, gidx3).reshape(N)


def kernel(sem_logits, source_pos, target_pos, conv_w, conv_b):
    B, K, Ms = sem_logits.shape
    Nt = target_pos.shape[2]
    wb = (conv_w[0, :, 0] + conv_b[0]).reshape(K, 1)
    gidx_flat, sval_flat = _tc_stage(sem_logits, source_pos, target_pos, wb,
                                     tn=4096)
    gate_flat = _sc_gather(sval_flat, gidx_flat)
    return gate_flat.reshape(B, 1, Nt)
